# baseline (speedup 1.0000x reference)
import functools

import jax
import jax.numpy as jnp
from jax import lax
from jax.experimental import pallas as pl
from jax.experimental.pallas import tpu as pltpu
from jax.experimental.pallas import tpu_sc as plsc

N = 10000
E = 320000
IN_C = 128
OUT_C = 16
HEADS = 8
HC = HEADS * OUT_C
NEG_SLOPE = 0.2
ROW = HC + 16

NC = 2
NS = 16
LANES = 16
EPW = E // (NC * NS)
CHUNK = 80
NCHUNKS = EPW // CHUNK
NPAD = 10240
RPT = NPAD // NS
PIECE = 80


def _lrelu(v):
    return jnp.maximum(v, 0.0) + NEG_SLOPE * jnp.minimum(v, 0.0)


def _s2():
    row = lax.broadcasted_iota(jnp.int32, (HC, 2 * HEADS), 0) // OUT_C
    col = lax.broadcasted_iota(jnp.int32, (HC, 2 * HEADS), 1) % HEADS
    return (row == col).astype(jnp.float32)


def _project_body(x_ref, w_ref, asrc_ref, adst_ref, hs_ref, ad_ref):
    h = jnp.dot(x_ref[...], w_ref[...], preferred_element_type=jnp.float32)
    s2 = _s2()
    hs_ref[:, :HC] = h
    hs_ref[:, HC:ROW] = jnp.dot(h * asrc_ref[...], s2,
                                preferred_element_type=jnp.float32)
    ad_ref[...] = jnp.dot(h * adst_ref[...], s2,
                          preferred_element_type=jnp.float32)


def _project(x, w, asrc_flat, adst_flat):
    blk = 2000
    return pl.pallas_call(
        _project_body,
        grid=(N // blk,),
        in_specs=[
            pl.BlockSpec((blk, IN_C), lambda i: (i, 0)),
            pl.BlockSpec((IN_C, HC), lambda i: (0, 0)),
            pl.BlockSpec((1, HC), lambda i: (0, 0)),
            pl.BlockSpec((1, HC), lambda i: (0, 0)),
        ],
        out_specs=[
            pl.BlockSpec((blk, ROW), lambda i: (i, 0)),
            pl.BlockSpec((blk, 2 * HEADS), lambda i: (i, 0)),
        ],
        out_shape=[
            jax.ShapeDtypeStruct((N, ROW), jnp.float32),
            jax.ShapeDtypeStruct((N, 2 * HEADS), jnp.float32),
        ],
    )(x, w, asrc_flat, adst_flat)


def _splat(v, b):
    idx = jnp.full((LANES, 1), b, dtype=jnp.int32)
    dn = lax.GatherDimensionNumbers(
        offset_dims=(), collapsed_slice_dims=(0,), start_index_map=(0,))
    return lax.gather(v, idx, dn, slice_sizes=(1,),
                      mode=lax.GatherScatterMode.PROMISE_IN_BOUNDS)


def _edge_body(src_hbm, dst_hbm, hs_hbm, ad_hbm, acc_hbm,
               sidx0, sidx1, sidx2, didx0, didx1, didx2,
               hsb0, hsb1, hsb2, adb0, adb1, adb2, acc_sh,
               si0, si1, si2, sh0, sh1, sh2, sa0, sa1, sa2):
    cid = lax.axis_index("c")
    sid = lax.axis_index("s")
    sidx = (sidx0, sidx1, sidx2)
    didx = (didx0, didx1, didx2)
    hsb = (hsb0, hsb1, hsb2)
    adb = (adb0, adb1, adb2)
    sem_i = (si0, si1, si2)
    sem_h = (sh0, sh1, sh2)
    sem_a = (sa0, sa1, sa2)

    def _zrow(r, carry):
        for j in range(ROW // LANES):
            hsb0[r, pl.ds(j * LANES, LANES)] = jnp.zeros((LANES,), jnp.float32)
        return carry
    lax.fori_loop(0, PIECE, _zrow, 0)

    base = cid * (E // NC) + sid * EPW

    def _fire_i(k, s):
        off = base + k * CHUNK
        pltpu.async_copy(src_hbm.at[pl.ds(off, CHUNK)], sidx[s], sem_i[s])
        pltpu.async_copy(dst_hbm.at[pl.ds(off, CHUNK)], didx[s], sem_i[s])

    def _wait_i(s):
        pltpu.make_async_copy(src_hbm.at[pl.ds(0, CHUNK)], sidx[s], sem_i[s]).wait()
        pltpu.make_async_copy(dst_hbm.at[pl.ds(0, CHUNK)], didx[s], sem_i[s]).wait()

    def _fire_g(s):
        pltpu.async_copy(hs_hbm.at[sidx[s]], hsb[s], sem_h[s])
        pltpu.async_copy(ad_hbm.at[didx[s]], adb[s], sem_a[s])

    def _wait_g(s):
        pltpu.make_async_copy(hs_hbm.at[sidx[s]], hsb[s], sem_h[s]).wait()
        pltpu.make_async_copy(ad_hbm.at[didx[s]], adb[s], sem_a[s]).wait()

    def _fire_sc(s):
        pltpu.sync_copy(hsb[s], acc_sh.at[didx[s]], add=True)

    def _compute(s):
        h_b, a_b = hsb[s], adb[s]

        @plsc.parallel_loop(0, CHUNK, 1, unroll=3)
        def _edge_scale(e):
            ee = _lrelu(h_b[e, pl.ds(HC, LANES)] + a_b[e, :])
            p16 = jnp.exp(ee)
            h_b[e, pl.ds(HC, LANES)] = p16
            for b in range(HEADS):
                pb = _splat(p16, b)
                hv = h_b[e, pl.ds(b * OUT_C, OUT_C)]
                h_b[e, pl.ds(b * OUT_C, OUT_C)] = hv * pb

    for piece in range(RPT // PIECE):
        pltpu.sync_copy(hsb0, acc_sh.at[pl.ds(sid * RPT + piece * PIECE, PIECE)])
    plsc.subcore_barrier()

    _fire_i(0, 0)
    _fire_i(1, 1)
    _wait_i(0)
    _fire_g(0)
    _fire_i(2, 2)
    _wait_i(1)
    _fire_g(1)
    _wait_g(0)
    _compute(0)
    _fire_sc(0)

    def _steady(k, s):
        _fire_i(k + 2, (s + 2) % 3)
        _wait_i((s + 1) % 3)
        _fire_g((s + 1) % 3)
        _wait_g(s)
        _compute(s)
        _fire_sc(s)

    def _triple(j, carry):
        k = 3 * j + 1
        _steady(k, 1)
        _steady(k + 1, 2)
        _steady(k + 2, 0)
        return carry

    lax.fori_loop(0, 40, _triple, 0)

    _fire_i(123, 0); _wait_i(2); _fire_g(2)
    _wait_g(1); _compute(1); _fire_sc(1)
    _fire_i(124, 1); _wait_i(0); _fire_g(0)
    _wait_g(2); _compute(2); _fire_sc(2)
    _wait_i(1); _fire_g(1)
    _wait_g(0); _compute(0); _fire_sc(0)
    _wait_g(1); _compute(1); _fire_sc(1)
    plsc.subcore_barrier()

    for piece in range(RPT // PIECE):
        r0 = sid * RPT + piece * PIECE
        pltpu.sync_copy(acc_sh.at[pl.ds(r0, PIECE)], hsb0)
        pltpu.sync_copy(hsb0, acc_hbm.at[cid, pl.ds(r0, PIECE)])


def _edge_pass(src, dst, hs, ad):
    mesh = plsc.VectorSubcoreMesh(core_axis_name="c", subcore_axis_name="s")
    kern = functools.partial(
        pl.kernel,
        mesh=mesh,
        compiler_params=pltpu.CompilerParams(use_tc_tiling_on_sc=False),
        out_type=jax.ShapeDtypeStruct((NC, NPAD, ROW), jnp.float32),
        scratch_types=[
            pltpu.VMEM((CHUNK,), jnp.int32),
            pltpu.VMEM((CHUNK,), jnp.int32),
            pltpu.VMEM((CHUNK,), jnp.int32),
            pltpu.VMEM((CHUNK,), jnp.int32),
            pltpu.VMEM((CHUNK,), jnp.int32),
            pltpu.VMEM((CHUNK,), jnp.int32),
            pltpu.VMEM((CHUNK, ROW), jnp.float32),
            pltpu.VMEM((CHUNK, ROW), jnp.float32),
            pltpu.VMEM((CHUNK, ROW), jnp.float32),
            pltpu.VMEM((CHUNK, LANES), jnp.float32),
            pltpu.VMEM((CHUNK, LANES), jnp.float32),
            pltpu.VMEM((CHUNK, LANES), jnp.float32),
            pltpu.VMEM_SHARED((NPAD, ROW), jnp.float32),
            pltpu.SemaphoreType.DMA,
            pltpu.SemaphoreType.DMA,
            pltpu.SemaphoreType.DMA,
            pltpu.SemaphoreType.DMA,
            pltpu.SemaphoreType.DMA,
            pltpu.SemaphoreType.DMA,
            pltpu.SemaphoreType.DMA,
            pltpu.SemaphoreType.DMA,
            pltpu.SemaphoreType.DMA,
        ],
    )(_edge_body)
    return kern(src, dst, hs, ad)


def _finalize_body(acc_ref, hs_ref, ad_ref, bias_ref, out_ref):
    a = acc_ref[0] + acc_ref[1]
    p16 = jnp.exp(_lrelu(hs_ref[:, HC:ROW] + ad_ref[...]))
    p8 = p16[:, :HEADS]
    row = lax.broadcasted_iota(jnp.int32, (HEADS, HC), 0)
    col = lax.broadcasted_iota(jnp.int32, (HEADS, HC), 1) // OUT_C
    t = (row == col).astype(jnp.float32)
    prep = jnp.dot(p8, t, preferred_element_type=jnp.float32)
    num = a[:, :HC] + hs_ref[:, :HC] * prep
    den = a[:, HC:HC + HEADS] + p8
    drep = jnp.dot(den, t, preferred_element_type=jnp.float32) + 1e-16
    out_ref[...] = num / drep + bias_ref[...]


def _finalize(acc, hs, ad, bias2d):
    blk = 2000
    return pl.pallas_call(
        _finalize_body,
        grid=(N // blk,),
        in_specs=[
            pl.BlockSpec((NC, blk, ROW), lambda i: (0, i, 0)),
            pl.BlockSpec((blk, ROW), lambda i: (i, 0)),
            pl.BlockSpec((blk, 2 * HEADS), lambda i: (i, 0)),
            pl.BlockSpec((1, HC), lambda i: (0, 0)),
        ],
        out_specs=pl.BlockSpec((blk, HC), lambda i: (i, 0)),
        out_shape=jax.ShapeDtypeStruct((N, HC), jnp.float32),
    )(acc, hs, ad, bias2d)


def kernel(x, edge_index, W, att_src, att_dst, bias):
    src = edge_index[0]
    dst = edge_index[1]
    hs, ad = _project(x, W, att_src.reshape(1, HC), att_dst.reshape(1, HC))
    acc = _edge_pass(src, dst, hs, ad)
    out = _finalize(acc, hs, ad, bias.reshape(1, HC))
    return out


# speedup vs baseline: 1.0222x; 1.0222x over previous
import functools

import jax
import jax.numpy as jnp
from jax import lax
from jax.experimental import pallas as pl
from jax.experimental.pallas import tpu as pltpu
from jax.experimental.pallas import tpu_sc as plsc

N = 10000
E = 320000
IN_C = 128
OUT_C = 16
HEADS = 8
HC = HEADS * OUT_C
NEG_SLOPE = 0.2
ROW = HC + 16

NC = 2
NS = 16
LANES = 16
EPW = E // (NC * NS)
CHUNK = 80
NCHUNKS = EPW // CHUNK
NPAD = 10240
RPT = NPAD // NS
PIECE = 80


def _lrelu(v):
    return jnp.maximum(v, 0.0) + NEG_SLOPE * jnp.minimum(v, 0.0)


def _s2():
    row = lax.broadcasted_iota(jnp.int32, (HC, 2 * HEADS), 0) // OUT_C
    col = lax.broadcasted_iota(jnp.int32, (HC, 2 * HEADS), 1) % HEADS
    return (row == col).astype(jnp.float32)


def _project_body(x_ref, w_ref, asrc_ref, adst_ref, hs_ref, ad_ref):
    h = jnp.dot(x_ref[...], w_ref[...], preferred_element_type=jnp.float32)
    s2 = _s2()
    hs_ref[:, :HC] = h
    hs_ref[:, HC:ROW] = jnp.dot(h * asrc_ref[...], s2,
                                preferred_element_type=jnp.float32)
    ad_ref[...] = jnp.dot(h * adst_ref[...], s2,
                          preferred_element_type=jnp.float32)


def _project(x, w, asrc_flat, adst_flat):
    blk = 2000
    return pl.pallas_call(
        _project_body,
        grid=(N // blk,),
        in_specs=[
            pl.BlockSpec((blk, IN_C), lambda i: (i, 0)),
            pl.BlockSpec((IN_C, HC), lambda i: (0, 0)),
            pl.BlockSpec((1, HC), lambda i: (0, 0)),
            pl.BlockSpec((1, HC), lambda i: (0, 0)),
        ],
        out_specs=[
            pl.BlockSpec((blk, ROW), lambda i: (i, 0)),
            pl.BlockSpec((blk, 2 * HEADS), lambda i: (i, 0)),
        ],
        out_shape=[
            jax.ShapeDtypeStruct((N, ROW), jnp.float32),
            jax.ShapeDtypeStruct((N, 2 * HEADS), jnp.float32),
        ],
    )(x, w, asrc_flat, adst_flat)


def _splat(v, b):
    idx = jnp.full((LANES, 1), b, dtype=jnp.int32)
    dn = lax.GatherDimensionNumbers(
        offset_dims=(), collapsed_slice_dims=(0,), start_index_map=(0,))
    return lax.gather(v, idx, dn, slice_sizes=(1,),
                      mode=lax.GatherScatterMode.PROMISE_IN_BOUNDS)


def _edge_body(src_hbm, dst_hbm, hs_hbm, ad_hbm, acc_hbm,
               sidx0, sidx1, sidx2, didx0, didx1, didx2,
               hsb0, hsb1, hsb2, adb0, adb1, adb2, acc_sh,
               si0, si1, si2, sh0, sh1, sh2, sa0, sa1, sa2):
    cid = lax.axis_index("c")
    sid = lax.axis_index("s")
    sidx = (sidx0, sidx1, sidx2)
    didx = (didx0, didx1, didx2)
    hsb = (hsb0, hsb1, hsb2)
    adb = (adb0, adb1, adb2)
    sem_i = (si0, si1, si2)
    sem_h = (sh0, sh1, sh2)
    sem_a = (sa0, sa1, sa2)

    def _zrow(r, carry):
        for j in range(ROW // LANES):
            hsb0[r, pl.ds(j * LANES, LANES)] = jnp.zeros((LANES,), jnp.float32)
        return carry
    lax.fori_loop(0, PIECE, _zrow, 0)

    base = cid * (E // NC) + sid * EPW

    def _fire_i(k, s):
        off = base + k * CHUNK
        pltpu.async_copy(src_hbm.at[pl.ds(off, CHUNK)], sidx[s], sem_i[s])
        pltpu.async_copy(dst_hbm.at[pl.ds(off, CHUNK)], didx[s], sem_i[s])

    def _wait_i(s):
        pltpu.make_async_copy(src_hbm.at[pl.ds(0, CHUNK)], sidx[s], sem_i[s]).wait()
        pltpu.make_async_copy(dst_hbm.at[pl.ds(0, CHUNK)], didx[s], sem_i[s]).wait()

    def _fire_g(s):
        pltpu.async_copy(hs_hbm.at[sidx[s]], hsb[s], sem_h[s])
        pltpu.async_copy(ad_hbm.at[didx[s]], adb[s], sem_a[s])

    def _wait_g(s):
        pltpu.make_async_copy(hs_hbm.at[sidx[s]], hsb[s], sem_h[s]).wait()
        pltpu.make_async_copy(ad_hbm.at[didx[s]], adb[s], sem_a[s]).wait()

    def _fire_sc(s):
        pltpu.sync_copy(hsb[s], acc_sh.at[didx[s]], add=True)

    def _compute(s):
        h_b, a_b = hsb[s], adb[s]

        @plsc.parallel_loop(0, CHUNK, 1, unroll=2)
        def _edge_scale(e):
            ee = _lrelu(h_b[e, pl.ds(HC, LANES)] + a_b[e, :])
            p16 = jnp.exp(ee)
            h_b[e, pl.ds(HC, LANES)] = p16
            for b in range(HEADS):
                pb = _splat(p16, b)
                hv = h_b[e, pl.ds(b * OUT_C, OUT_C)]
                h_b[e, pl.ds(b * OUT_C, OUT_C)] = hv * pb

    for piece in range(RPT // PIECE):
        pltpu.sync_copy(hsb0, acc_sh.at[pl.ds(sid * RPT + piece * PIECE, PIECE)])
    plsc.subcore_barrier()

    _fire_i(0, 0)
    _fire_i(1, 1)
    _wait_i(0)
    _fire_g(0)
    _fire_i(2, 2)
    _wait_i(1)
    _fire_g(1)
    _wait_g(0)
    _compute(0)
    _fire_sc(0)

    def _steady(k, s):
        _fire_i(k + 2, (s + 2) % 3)
        _wait_i((s + 1) % 3)
        _fire_g((s + 1) % 3)
        _wait_g(s)
        _compute(s)
        _fire_sc(s)

    def _triple(j, carry):
        k = 3 * j + 1
        _steady(k, 1)
        _steady(k + 1, 2)
        _steady(k + 2, 0)
        return carry

    lax.fori_loop(0, 40, _triple, 0)

    _fire_i(123, 0); _wait_i(2); _fire_g(2)
    _wait_g(1); _compute(1); _fire_sc(1)
    _fire_i(124, 1); _wait_i(0); _fire_g(0)
    _wait_g(2); _compute(2); _fire_sc(2)
    _wait_i(1); _fire_g(1)
    _wait_g(0); _compute(0); _fire_sc(0)
    _wait_g(1); _compute(1); _fire_sc(1)
    plsc.subcore_barrier()

    for piece in range(RPT // PIECE):
        r0 = sid * RPT + piece * PIECE
        pltpu.sync_copy(acc_sh.at[pl.ds(r0, PIECE)], hsb0)
        pltpu.sync_copy(hsb0, acc_hbm.at[cid, pl.ds(r0, PIECE)])


def _edge_pass(src, dst, hs, ad):
    mesh = plsc.VectorSubcoreMesh(core_axis_name="c", subcore_axis_name="s")
    kern = functools.partial(
        pl.kernel,
        mesh=mesh,
        compiler_params=pltpu.CompilerParams(use_tc_tiling_on_sc=False),
        out_type=jax.ShapeDtypeStruct((NC, NPAD, ROW), jnp.float32),
        scratch_types=[
            pltpu.VMEM((CHUNK,), jnp.int32),
            pltpu.VMEM((CHUNK,), jnp.int32),
            pltpu.VMEM((CHUNK,), jnp.int32),
            pltpu.VMEM((CHUNK,), jnp.int32),
            pltpu.VMEM((CHUNK,), jnp.int32),
            pltpu.VMEM((CHUNK,), jnp.int32),
            pltpu.VMEM((CHUNK, ROW), jnp.float32),
            pltpu.VMEM((CHUNK, ROW), jnp.float32),
            pltpu.VMEM((CHUNK, ROW), jnp.float32),
            pltpu.VMEM((CHUNK, LANES), jnp.float32),
            pltpu.VMEM((CHUNK, LANES), jnp.float32),
            pltpu.VMEM((CHUNK, LANES), jnp.float32),
            pltpu.VMEM_SHARED((NPAD, ROW), jnp.float32),
            pltpu.SemaphoreType.DMA,
            pltpu.SemaphoreType.DMA,
            pltpu.SemaphoreType.DMA,
            pltpu.SemaphoreType.DMA,
            pltpu.SemaphoreType.DMA,
            pltpu.SemaphoreType.DMA,
            pltpu.SemaphoreType.DMA,
            pltpu.SemaphoreType.DMA,
            pltpu.SemaphoreType.DMA,
        ],
    )(_edge_body)
    return kern(src, dst, hs, ad)


def _finalize_body(acc_ref, hs_ref, ad_ref, bias_ref, out_ref):
    a = acc_ref[0] + acc_ref[1]
    p16 = jnp.exp(_lrelu(hs_ref[:, HC:ROW] + ad_ref[...]))
    p8 = p16[:, :HEADS]
    row = lax.broadcasted_iota(jnp.int32, (HEADS, HC), 0)
    col = lax.broadcasted_iota(jnp.int32, (HEADS, HC), 1) // OUT_C
    t = (row == col).astype(jnp.float32)
    prep = jnp.dot(p8, t, preferred_element_type=jnp.float32)
    num = a[:, :HC] + hs_ref[:, :HC] * prep
    den = a[:, HC:HC + HEADS] + p8
    drep = jnp.dot(den, t, preferred_element_type=jnp.float32) + 1e-16
    out_ref[...] = num / drep + bias_ref[...]


def _finalize(acc, hs, ad, bias2d):
    blk = 2000
    return pl.pallas_call(
        _finalize_body,
        grid=(N // blk,),
        in_specs=[
            pl.BlockSpec((NC, blk, ROW), lambda i: (0, i, 0)),
            pl.BlockSpec((blk, ROW), lambda i: (i, 0)),
            pl.BlockSpec((blk, 2 * HEADS), lambda i: (i, 0)),
            pl.BlockSpec((1, HC), lambda i: (0, 0)),
        ],
        out_specs=pl.BlockSpec((blk, HC), lambda i: (i, 0)),
        out_shape=jax.ShapeDtypeStruct((N, HC), jnp.float32),
    )(acc, hs, ad, bias2d)


def kernel(x, edge_index, W, att_src, att_dst, bias):
    src = edge_index[0]
    dst = edge_index[1]
    hs, ad = _project(x, W, att_src.reshape(1, HC), att_dst.reshape(1, HC))
    acc = _edge_pass(src, dst, hs, ad)
    out = _finalize(acc, hs, ad, bias.reshape(1, HC))
    return out


# speedup vs baseline: 1.0898x; 1.0661x over previous
import functools

import jax
import jax.numpy as jnp
from jax import lax
from jax.experimental import pallas as pl
from jax.experimental.pallas import tpu as pltpu
from jax.experimental.pallas import tpu_sc as plsc

N = 10000
E = 320000
IN_C = 128
OUT_C = 16
HEADS = 8
HC = HEADS * OUT_C
NEG_SLOPE = 0.2

NC = 2
NS = 16
LANES = 16
EPW = E // (NC * NS)
CHUNK = 80
NCHUNKS = EPW // CHUNK
NPAD = 10240
RPT = NPAD // NS
PIECE = 80


def _lrelu(v):
    return jnp.maximum(v, 0.0) + NEG_SLOPE * jnp.minimum(v, 0.0)


def _s2():
    row = lax.broadcasted_iota(jnp.int32, (HC, 2 * HEADS), 0) // OUT_C
    col = lax.broadcasted_iota(jnp.int32, (HC, 2 * HEADS), 1) % HEADS
    return (row == col).astype(jnp.float32)


def _project_body(x_ref, w_ref, asrc_ref, adst_ref, h_ref, as_ref, ad_ref):
    h = jnp.dot(x_ref[...], w_ref[...], preferred_element_type=jnp.float32)
    s2 = _s2()
    h_ref[...] = h
    as_ref[...] = jnp.dot(h * asrc_ref[...], s2,
                          preferred_element_type=jnp.float32)
    ad_ref[...] = jnp.dot(h * adst_ref[...], s2,
                          preferred_element_type=jnp.float32)


def _project(x, w, asrc_flat, adst_flat):
    blk = 2000
    return pl.pallas_call(
        _project_body,
        grid=(N // blk,),
        in_specs=[
            pl.BlockSpec((blk, IN_C), lambda i: (i, 0)),
            pl.BlockSpec((IN_C, HC), lambda i: (0, 0)),
            pl.BlockSpec((1, HC), lambda i: (0, 0)),
            pl.BlockSpec((1, HC), lambda i: (0, 0)),
        ],
        out_specs=[
            pl.BlockSpec((blk, HC), lambda i: (i, 0)),
            pl.BlockSpec((blk, 2 * HEADS), lambda i: (i, 0)),
            pl.BlockSpec((blk, 2 * HEADS), lambda i: (i, 0)),
        ],
        out_shape=[
            jax.ShapeDtypeStruct((N, HC), jnp.float32),
            jax.ShapeDtypeStruct((N, 2 * HEADS), jnp.float32),
            jax.ShapeDtypeStruct((N, 2 * HEADS), jnp.float32),
        ],
    )(x, w, asrc_flat, adst_flat)


def _splat(v, b):
    idx = jnp.full((LANES, 1), b, dtype=jnp.int32)
    dn = lax.GatherDimensionNumbers(
        offset_dims=(), collapsed_slice_dims=(0,), start_index_map=(0,))
    return lax.gather(v, idx, dn, slice_sizes=(1,),
                      mode=lax.GatherScatterMode.PROMISE_IN_BOUNDS)


def _edge_body(src_hbm, dst_hbm, h_hbm, as_hbm, ad_hbm, msg_hbm, den_hbm,
               sidx0, sidx1, sidx2, didx0, didx1, didx2,
               hb0, hb1, hb2, asb0, asb1, asb2, adb0, adb1, adb2,
               msg_sh, den_sh,
               si0, si1, si2, sh0, sh1, sh2, sb0, sb1, sb2, sa0, sa1, sa2):
    cid = lax.axis_index("c")
    sid = lax.axis_index("s")
    sidx = (sidx0, sidx1, sidx2)
    didx = (didx0, didx1, didx2)
    hb = (hb0, hb1, hb2)
    asb = (asb0, asb1, asb2)
    adb = (adb0, adb1, adb2)
    sem_i = (si0, si1, si2)
    sem_h = (sh0, sh1, sh2)
    sem_b = (sb0, sb1, sb2)
    sem_a = (sa0, sa1, sa2)

    def _zrow(r, carry):
        for j in range(HC // LANES):
            hb0[r, pl.ds(j * LANES, LANES)] = jnp.zeros((LANES,), jnp.float32)
        asb0[r, :] = jnp.zeros((LANES,), jnp.float32)
        return carry
    lax.fori_loop(0, PIECE, _zrow, 0)
    for piece in range(RPT // PIECE):
        r0 = sid * RPT + piece * PIECE
        pltpu.sync_copy(hb0, msg_sh.at[pl.ds(r0, PIECE)])
        pltpu.sync_copy(asb0, den_sh.at[pl.ds(r0, PIECE)])
    plsc.subcore_barrier()

    base = cid * (E // NC) + sid * EPW

    def _fire_i(k, s):
        off = base + k * CHUNK
        pltpu.async_copy(src_hbm.at[pl.ds(off, CHUNK)], sidx[s], sem_i[s])
        pltpu.async_copy(dst_hbm.at[pl.ds(off, CHUNK)], didx[s], sem_i[s])

    def _wait_i(s):
        pltpu.make_async_copy(src_hbm.at[pl.ds(0, CHUNK)], sidx[s], sem_i[s]).wait()
        pltpu.make_async_copy(dst_hbm.at[pl.ds(0, CHUNK)], didx[s], sem_i[s]).wait()

    def _fire_g(s):
        pltpu.async_copy(h_hbm.at[sidx[s]], hb[s], sem_h[s])
        pltpu.async_copy(as_hbm.at[sidx[s]], asb[s], sem_b[s])
        pltpu.async_copy(ad_hbm.at[didx[s]], adb[s], sem_a[s])

    def _wait_g(s):
        pltpu.make_async_copy(h_hbm.at[sidx[s]], hb[s], sem_h[s]).wait()
        pltpu.make_async_copy(as_hbm.at[sidx[s]], asb[s], sem_b[s]).wait()
        pltpu.make_async_copy(ad_hbm.at[didx[s]], adb[s], sem_a[s]).wait()

    def _fire_sc(s):
        pltpu.sync_copy(hb[s], msg_sh.at[didx[s]], add=True)
        pltpu.sync_copy(asb[s], den_sh.at[didx[s]], add=True)

    def _compute(s):
        h_b, s_b, a_b = hb[s], asb[s], adb[s]

        @plsc.parallel_loop(0, CHUNK, 1, unroll=2)
        def _edge_scale(e):
            ee = _lrelu(s_b[e, :] + a_b[e, :])
            p16 = jnp.exp(ee)
            s_b[e, :] = p16
            for b in range(HEADS):
                pb = _splat(p16, b)
                hv = h_b[e, pl.ds(b * OUT_C, OUT_C)]
                h_b[e, pl.ds(b * OUT_C, OUT_C)] = hv * pb

    _fire_i(0, 0)
    _fire_i(1, 1)
    _wait_i(0)
    _fire_g(0)
    _fire_i(2, 2)
    _wait_i(1)
    _fire_g(1)
    _wait_g(0)
    _compute(0)
    _fire_sc(0)

    def _steady(k, s):
        _fire_i(k + 2, (s + 2) % 3)
        _wait_i((s + 1) % 3)
        _fire_g((s + 1) % 3)
        _wait_g(s)
        _compute(s)
        _fire_sc(s)

    def _triple(j, carry):
        k = 3 * j + 1
        _steady(k, 1)
        _steady(k + 1, 2)
        _steady(k + 2, 0)
        return carry

    lax.fori_loop(0, 40, _triple, 0)

    _fire_i(123, 0); _wait_i(2); _fire_g(2)
    _wait_g(1); _compute(1); _fire_sc(1)
    _fire_i(124, 1); _wait_i(0); _fire_g(0)
    _wait_g(2); _compute(2); _fire_sc(2)
    _wait_i(1); _fire_g(1)
    _wait_g(0); _compute(0); _fire_sc(0)
    _wait_g(1); _compute(1); _fire_sc(1)
    plsc.subcore_barrier()

    for piece in range(RPT // PIECE):
        r0 = sid * RPT + piece * PIECE
        pltpu.sync_copy(msg_sh.at[pl.ds(r0, PIECE)], hb0)
        pltpu.sync_copy(hb0, msg_hbm.at[cid, pl.ds(r0, PIECE)])
        pltpu.sync_copy(den_sh.at[pl.ds(r0, PIECE)], asb0)
        pltpu.sync_copy(asb0, den_hbm.at[cid, pl.ds(r0, PIECE)])


def _edge_pass(src, dst, h, as2, ad2):
    mesh = plsc.VectorSubcoreMesh(core_axis_name="c", subcore_axis_name="s")
    kern = functools.partial(
        pl.kernel,
        mesh=mesh,
        compiler_params=pltpu.CompilerParams(use_tc_tiling_on_sc=False),
        out_type=[
            jax.ShapeDtypeStruct((NC, NPAD, HC), jnp.float32),
            jax.ShapeDtypeStruct((NC, NPAD, LANES), jnp.float32),
        ],
        scratch_types=[
            pltpu.VMEM((CHUNK,), jnp.int32),
            pltpu.VMEM((CHUNK,), jnp.int32),
            pltpu.VMEM((CHUNK,), jnp.int32),
            pltpu.VMEM((CHUNK,), jnp.int32),
            pltpu.VMEM((CHUNK,), jnp.int32),
            pltpu.VMEM((CHUNK,), jnp.int32),
            pltpu.VMEM((CHUNK, HC), jnp.float32),
            pltpu.VMEM((CHUNK, HC), jnp.float32),
            pltpu.VMEM((CHUNK, HC), jnp.float32),
            pltpu.VMEM((CHUNK, LANES), jnp.float32),
            pltpu.VMEM((CHUNK, LANES), jnp.float32),
            pltpu.VMEM((CHUNK, LANES), jnp.float32),
            pltpu.VMEM((CHUNK, LANES), jnp.float32),
            pltpu.VMEM((CHUNK, LANES), jnp.float32),
            pltpu.VMEM((CHUNK, LANES), jnp.float32),
            pltpu.VMEM_SHARED((NPAD, HC), jnp.float32),
            pltpu.VMEM_SHARED((NPAD, LANES), jnp.float32),
            pltpu.SemaphoreType.DMA,
            pltpu.SemaphoreType.DMA,
            pltpu.SemaphoreType.DMA,
            pltpu.SemaphoreType.DMA,
            pltpu.SemaphoreType.DMA,
            pltpu.SemaphoreType.DMA,
            pltpu.SemaphoreType.DMA,
            pltpu.SemaphoreType.DMA,
            pltpu.SemaphoreType.DMA,
            pltpu.SemaphoreType.DMA,
            pltpu.SemaphoreType.DMA,
            pltpu.SemaphoreType.DMA,
        ],
    )(_edge_body)
    return kern(src, dst, h, as2, ad2)


def _finalize_body(msg_ref, den_ref, h_ref, as_ref, ad_ref, bias_ref, out_ref):
    a = msg_ref[0] + msg_ref[1]
    d16 = den_ref[0] + den_ref[1]
    p16 = jnp.exp(_lrelu(as_ref[...] + ad_ref[...]))
    p8 = p16[:, :HEADS]
    row = lax.broadcasted_iota(jnp.int32, (HEADS, HC), 0)
    col = lax.broadcasted_iota(jnp.int32, (HEADS, HC), 1) // OUT_C
    t = (row == col).astype(jnp.float32)
    prep = jnp.dot(p8, t, preferred_element_type=jnp.float32)
    num = a + h_ref[...] * prep
    den = d16[:, :HEADS] + p8
    drep = jnp.dot(den, t, preferred_element_type=jnp.float32) + 1e-16
    out_ref[...] = num / drep + bias_ref[...]


def _finalize(msg, den, h, as2, ad2, bias2d):
    blk = 2000
    return pl.pallas_call(
        _finalize_body,
        grid=(N // blk,),
        in_specs=[
            pl.BlockSpec((NC, blk, HC), lambda i: (0, i, 0)),
            pl.BlockSpec((NC, blk, LANES), lambda i: (0, i, 0)),
            pl.BlockSpec((blk, HC), lambda i: (i, 0)),
            pl.BlockSpec((blk, 2 * HEADS), lambda i: (i, 0)),
            pl.BlockSpec((blk, 2 * HEADS), lambda i: (i, 0)),
            pl.BlockSpec((1, HC), lambda i: (0, 0)),
        ],
        out_specs=pl.BlockSpec((blk, HC), lambda i: (i, 0)),
        out_shape=jax.ShapeDtypeStruct((N, HC), jnp.float32),
    )(msg, den, h, as2, ad2, bias2d)


def kernel(x, edge_index, W, att_src, att_dst, bias):
    src = edge_index[0]
    dst = edge_index[1]
    h, as2, ad2 = _project(x, W, att_src.reshape(1, HC), att_dst.reshape(1, HC))
    msg, den = _edge_pass(src, dst, h, as2, ad2)
    out = _finalize(msg, den, h, as2, ad2, bias.reshape(1, HC))
    return out


# speedup vs baseline: 1.0980x; 1.0075x over previous
import functools

import jax
import jax.numpy as jnp
from jax import lax
from jax.experimental import pallas as pl
from jax.experimental.pallas import tpu as pltpu
from jax.experimental.pallas import tpu_sc as plsc

N = 10000
E = 320000
IN_C = 128
OUT_C = 16
HEADS = 8
HC = HEADS * OUT_C
NEG_SLOPE = 0.2

NC = 2
NS = 16
LANES = 16
EPW = E // (NC * NS)
CHUNK = 80
NCHUNKS = EPW // CHUNK
NPAD = 10240
RPT = NPAD // NS
PIECE = 80


def _lrelu(v):
    return jnp.maximum(v, 0.0) + NEG_SLOPE * jnp.minimum(v, 0.0)


def _s2():
    row = lax.broadcasted_iota(jnp.int32, (HC, 2 * HEADS), 0) // OUT_C
    col = lax.broadcasted_iota(jnp.int32, (HC, 2 * HEADS), 1) % HEADS
    return (row == col).astype(jnp.float32)


def _project_body(x_ref, w_ref, asrc_ref, adst_ref, h_ref, as_ref, ad_ref):
    h = jnp.dot(x_ref[...], w_ref[...], preferred_element_type=jnp.float32)
    s2 = _s2()
    h_ref[...] = h
    as_ref[...] = jnp.dot(h * asrc_ref[...], s2,
                          preferred_element_type=jnp.float32)
    ad_ref[...] = jnp.dot(h * adst_ref[...], s2,
                          preferred_element_type=jnp.float32)


def _project(x, w, asrc_flat, adst_flat):
    blk = 2000
    return pl.pallas_call(
        _project_body,
        grid=(N // blk,),
        in_specs=[
            pl.BlockSpec((blk, IN_C), lambda i: (i, 0)),
            pl.BlockSpec((IN_C, HC), lambda i: (0, 0)),
            pl.BlockSpec((1, HC), lambda i: (0, 0)),
            pl.BlockSpec((1, HC), lambda i: (0, 0)),
        ],
        out_specs=[
            pl.BlockSpec((blk, HC), lambda i: (i, 0)),
            pl.BlockSpec((blk, 2 * HEADS), lambda i: (i, 0)),
            pl.BlockSpec((blk, 2 * HEADS), lambda i: (i, 0)),
        ],
        out_shape=[
            jax.ShapeDtypeStruct((N, HC), jnp.float32),
            jax.ShapeDtypeStruct((N, 2 * HEADS), jnp.float32),
            jax.ShapeDtypeStruct((N, 2 * HEADS), jnp.float32),
        ],
    )(x, w, asrc_flat, adst_flat)


def _splat(v, b):
    idx = jnp.full((LANES, 1), b, dtype=jnp.int32)
    dn = lax.GatherDimensionNumbers(
        offset_dims=(), collapsed_slice_dims=(0,), start_index_map=(0,))
    return lax.gather(v, idx, dn, slice_sizes=(1,),
                      mode=lax.GatherScatterMode.PROMISE_IN_BOUNDS)


def _edge_body(src_hbm, dst_hbm, h_hbm, as_hbm, ad_hbm, msg_hbm, den_hbm,
               sidx0, sidx1, sidx2, didx0, didx1, didx2,
               hb0, hb1, hb2, asb0, asb1, asb2, adb0, adb1, adb2,
               msg_sh, den_sh,
               si0, si1, si2, sh0, sh1, sh2, sb0, sb1, sb2, sa0, sa1, sa2):
    cid = lax.axis_index("c")
    sid = lax.axis_index("s")
    sidx = (sidx0, sidx1, sidx2)
    didx = (didx0, didx1, didx2)
    hb = (hb0, hb1, hb2)
    asb = (asb0, asb1, asb2)
    adb = (adb0, adb1, adb2)
    sem_i = (si0, si1, si2)
    sem_h = (sh0, sh1, sh2)
    sem_b = (sb0, sb1, sb2)
    sem_a = (sa0, sa1, sa2)

    def _zrow(r, carry):
        for j in range(HC // LANES):
            hb0[r, pl.ds(j * LANES, LANES)] = jnp.zeros((LANES,), jnp.float32)
        asb0[r, :] = jnp.zeros((LANES,), jnp.float32)
        return carry
    lax.fori_loop(0, PIECE, _zrow, 0)
    for piece in range(RPT // PIECE):
        r0 = sid * RPT + piece * PIECE
        pltpu.sync_copy(hb0, msg_sh.at[pl.ds(r0, PIECE)])
        pltpu.sync_copy(asb0, den_sh.at[pl.ds(r0, PIECE)])
    plsc.subcore_barrier()

    base = cid * (E // NC) + sid * EPW

    def _fire_i(k, s):
        off = base + k * CHUNK
        pltpu.async_copy(src_hbm.at[pl.ds(off, CHUNK)], sidx[s], sem_i[s])
        pltpu.async_copy(dst_hbm.at[pl.ds(off, CHUNK)], didx[s], sem_i[s])

    def _wait_i(s):
        pltpu.make_async_copy(src_hbm.at[pl.ds(0, CHUNK)], sidx[s], sem_i[s]).wait()
        pltpu.make_async_copy(dst_hbm.at[pl.ds(0, CHUNK)], didx[s], sem_i[s]).wait()

    def _fire_g(s):
        pltpu.async_copy(h_hbm.at[sidx[s]], hb[s], sem_h[s])
        pltpu.async_copy(as_hbm.at[sidx[s]], asb[s], sem_b[s])
        pltpu.async_copy(ad_hbm.at[didx[s]], adb[s], sem_a[s])

    def _wait_g(s):
        pltpu.make_async_copy(h_hbm.at[sidx[s]], hb[s], sem_h[s]).wait()
        pltpu.make_async_copy(as_hbm.at[sidx[s]], asb[s], sem_b[s]).wait()
        pltpu.make_async_copy(ad_hbm.at[didx[s]], adb[s], sem_a[s]).wait()

    def _fire_sc(s):
        pltpu.sync_copy(hb[s], msg_sh.at[didx[s]], add=True)
        pltpu.sync_copy(asb[s], den_sh.at[didx[s]], add=True)

    def _compute(s):
        h_b, s_b, a_b = hb[s], asb[s], adb[s]

        @plsc.parallel_loop(0, CHUNK, 1, unroll=2)
        def _edge_scale(e):
            ee = _lrelu(s_b[e, :] + a_b[e, :])
            p16 = jnp.exp(ee)
            s_b[e, :] = p16
            for b in range(HEADS):
                pb = _splat(p16, b)
                hv = h_b[e, pl.ds(b * OUT_C, OUT_C)]
                h_b[e, pl.ds(b * OUT_C, OUT_C)] = hv * pb

    _fire_i(0, 0)
    _fire_i(1, 1)
    _wait_i(0)
    _fire_g(0)
    _fire_i(2, 2)
    _wait_i(1)
    _fire_g(1)
    _wait_g(0)
    _compute(0)
    _fire_sc(0)

    def _steady(k, s):
        _fire_i(k + 2, (s + 2) % 3)
        _wait_i((s + 1) % 3)
        _fire_g((s + 1) % 3)
        _wait_g(s)
        _compute(s)
        _fire_sc(s)

    def _triple(j, carry):
        k = 3 * j + 1
        _steady(k, 1)
        _steady(k + 1, 2)
        _steady(k + 2, 0)
        return carry

    lax.fori_loop(0, 40, _triple, 0)

    _fire_i(123, 0); _wait_i(2); _fire_g(2)
    _wait_g(1); _compute(1); _fire_sc(1)
    _fire_i(124, 1); _wait_i(0); _fire_g(0)
    _wait_g(2); _compute(2); _fire_sc(2)
    _wait_i(1); _fire_g(1)
    _wait_g(0); _compute(0); _fire_sc(0)
    _wait_g(1); _compute(1); _fire_sc(1)
    plsc.subcore_barrier()

    r0 = sid * RPT
    pltpu.sync_copy(msg_sh.at[pl.ds(r0, RPT)], msg_hbm.at[cid, pl.ds(r0, RPT)])
    pltpu.sync_copy(den_sh.at[pl.ds(r0, RPT)], den_hbm.at[cid, pl.ds(r0, RPT)])


def _edge_pass(src, dst, h, as2, ad2):
    mesh = plsc.VectorSubcoreMesh(core_axis_name="c", subcore_axis_name="s")
    kern = functools.partial(
        pl.kernel,
        mesh=mesh,
        compiler_params=pltpu.CompilerParams(use_tc_tiling_on_sc=False),
        out_type=[
            jax.ShapeDtypeStruct((NC, NPAD, HC), jnp.float32),
            jax.ShapeDtypeStruct((NC, NPAD, LANES), jnp.float32),
        ],
        scratch_types=[
            pltpu.VMEM((CHUNK,), jnp.int32),
            pltpu.VMEM((CHUNK,), jnp.int32),
            pltpu.VMEM((CHUNK,), jnp.int32),
            pltpu.VMEM((CHUNK,), jnp.int32),
            pltpu.VMEM((CHUNK,), jnp.int32),
            pltpu.VMEM((CHUNK,), jnp.int32),
            pltpu.VMEM((CHUNK, HC), jnp.float32),
            pltpu.VMEM((CHUNK, HC), jnp.float32),
            pltpu.VMEM((CHUNK, HC), jnp.float32),
            pltpu.VMEM((CHUNK, LANES), jnp.float32),
            pltpu.VMEM((CHUNK, LANES), jnp.float32),
            pltpu.VMEM((CHUNK, LANES), jnp.float32),
            pltpu.VMEM((CHUNK, LANES), jnp.float32),
            pltpu.VMEM((CHUNK, LANES), jnp.float32),
            pltpu.VMEM((CHUNK, LANES), jnp.float32),
            pltpu.VMEM_SHARED((NPAD, HC), jnp.float32),
            pltpu.VMEM_SHARED((NPAD, LANES), jnp.float32),
            pltpu.SemaphoreType.DMA,
            pltpu.SemaphoreType.DMA,
            pltpu.SemaphoreType.DMA,
            pltpu.SemaphoreType.DMA,
            pltpu.SemaphoreType.DMA,
            pltpu.SemaphoreType.DMA,
            pltpu.SemaphoreType.DMA,
            pltpu.SemaphoreType.DMA,
            pltpu.SemaphoreType.DMA,
            pltpu.SemaphoreType.DMA,
            pltpu.SemaphoreType.DMA,
            pltpu.SemaphoreType.DMA,
        ],
    )(_edge_body)
    return kern(src, dst, h, as2, ad2)


def _finalize_body(msg_ref, den_ref, h_ref, as_ref, ad_ref, bias_ref, out_ref):
    a = msg_ref[0] + msg_ref[1]
    d16 = den_ref[0] + den_ref[1]
    p16 = jnp.exp(_lrelu(as_ref[...] + ad_ref[...]))
    p8 = p16[:, :HEADS]
    row = lax.broadcasted_iota(jnp.int32, (HEADS, HC), 0)
    col = lax.broadcasted_iota(jnp.int32, (HEADS, HC), 1) // OUT_C
    t = (row == col).astype(jnp.float32)
    prep = jnp.dot(p8, t, preferred_element_type=jnp.float32)
    num = a + h_ref[...] * prep
    den = d16[:, :HEADS] + p8
    drep = jnp.dot(den, t, preferred_element_type=jnp.float32) + 1e-16
    out_ref[...] = num / drep + bias_ref[...]


def _finalize(msg, den, h, as2, ad2, bias2d):
    blk = 2000
    return pl.pallas_call(
        _finalize_body,
        grid=(N // blk,),
        in_specs=[
            pl.BlockSpec((NC, blk, HC), lambda i: (0, i, 0)),
            pl.BlockSpec((NC, blk, LANES), lambda i: (0, i, 0)),
            pl.BlockSpec((blk, HC), lambda i: (i, 0)),
            pl.BlockSpec((blk, 2 * HEADS), lambda i: (i, 0)),
            pl.BlockSpec((blk, 2 * HEADS), lambda i: (i, 0)),
            pl.BlockSpec((1, HC), lambda i: (0, 0)),
        ],
        out_specs=pl.BlockSpec((blk, HC), lambda i: (i, 0)),
        out_shape=jax.ShapeDtypeStruct((N, HC), jnp.float32),
    )(msg, den, h, as2, ad2, bias2d)


def kernel(x, edge_index, W, att_src, att_dst, bias):
    src = edge_index[0]
    dst = edge_index[1]
    h, as2, ad2 = _project(x, W, att_src.reshape(1, HC), att_dst.reshape(1, HC))
    msg, den = _edge_pass(src, dst, h, as2, ad2)
    out = _finalize(msg, den, h, as2, ad2, bias.reshape(1, HC))
    return out
